# initial kernel scaffold (unmeasured)
import jax
import jax.numpy as jnp
from jax import lax
from jax.experimental import pallas as pl
from jax.experimental.pallas import tpu as pltpu

N_DEV = 4
SQ_SHARD = 256
SQ = 1024
D = 1024
HQ_LOCAL = 8
DH = 128
SKV = 4096
KV_CHUNK = 1024
N_KV_CHUNKS = SKV // KV_CHUNK
SCALE = 0.08838834764831843


def kernel(x, Wq, Wo, K_ext, V_ext):
    def body(
        x_ref, wq_ref, wo_ref, k_ref, v_ref, out_ref,
        xg_ref, kbuf_ref, vbuf_ref, partial_ref, rs_recv_ref,
        ag_send_sems, ag_recv_sems, rs_send_sems, rs_recv_sems, dma_sems,
    ):
        my = lax.axis_index("i")
        left = lax.rem(my + N_DEV - 1, N_DEV)
        right = lax.rem(my + 1, N_DEV)

        barrier_sem = pltpu.get_barrier_semaphore()
        for nbr in (left, right):
            pl.semaphore_signal(
                barrier_sem, inc=1,
                device_id=(nbr,), device_id_type=pl.DeviceIdType.MESH,
            )
        pl.semaphore_wait(barrier_sem, 2)

        pl.store(
            xg_ref,
            (pl.ds(my, 1), slice(None), slice(None)),
            x_ref[:, :, :],
        )
        for h in range(N_DEV - 1):
            src_slot = lax.rem(my - h + N_DEV, N_DEV)
            rdma = pltpu.make_async_remote_copy(
                src_ref=xg_ref.at[src_slot],
                dst_ref=xg_ref.at[src_slot],
                send_sem=ag_send_sems.at[h],
                recv_sem=ag_recv_sems.at[h],
                device_id=(right,),
                device_id_type=pl.DeviceIdType.MESH,
            )
            rdma.start()
            rdma.wait()

        xg = xg_ref[:, :, :].reshape(SQ, D)
        q = jnp.dot(xg, wq_ref[:, :], preferred_element_type=jnp.float32)

        h0 = my * HQ_LOCAL
        ms = [None] * HQ_LOCAL
        ls = [None] * HQ_LOCAL
        accs = [None] * HQ_LOCAL
        for c in range(N_KV_CHUNKS):
            copies = []
            for h in range(HQ_LOCAL):
                ck = pltpu.make_async_copy(
                    k_ref.at[0, pl.ds(c * KV_CHUNK, KV_CHUNK), h0 + h, :],
                    kbuf_ref.at[h],
                    dma_sems.at[0, h],
                )
                cv = pltpu.make_async_copy(
                    v_ref.at[0, pl.ds(c * KV_CHUNK, KV_CHUNK), h0 + h, :],
                    vbuf_ref.at[h],
                    dma_sems.at[1, h],
                )
                ck.start()
                cv.start()
                copies += [ck, cv]
            for cp in copies:
                cp.wait()
            for h in range(HQ_LOCAL):
                qh = q[:, h * DH:(h + 1) * DH]
                kh = kbuf_ref[h]
                vh = vbuf_ref[h]
                s = lax.dot_general(
                    qh, kh, (((1,), (1,)), ((), ())),
                    preferred_element_type=jnp.float32,
                ) * SCALE
                mj = jnp.max(s, axis=1, keepdims=True)
                p = jnp.exp(s - mj)
                lj = jnp.sum(p, axis=1, keepdims=True)
                pv = lax.dot_general(
                    p, vh, (((1,), (0,)), ((), ())),
                    preferred_element_type=jnp.float32,
                )
                if c == 0:
                    ms[h], ls[h], accs[h] = mj, lj, pv
                else:
                    m_new = jnp.maximum(ms[h], mj)
                    a_old = jnp.exp(ms[h] - m_new)
                    a_new = jnp.exp(mj - m_new)
                    ls[h] = ls[h] * a_old + lj * a_new
                    accs[h] = accs[h] * a_old + pv * a_new
                    ms[h] = m_new

        attn = jnp.concatenate(
            [accs[h] / ls[h] for h in range(HQ_LOCAL)], axis=1
        )

        partial = jnp.dot(attn, wo_ref[:, :], preferred_element_type=jnp.float32)
        partial_ref[:, :, :] = partial.reshape(N_DEV, SQ_SHARD, D)

        for s in range(N_DEV - 1):
            c = lax.rem(my - 1 - s + 2 * N_DEV, N_DEV)
            if s > 0:
                cur = pl.load(
                    partial_ref, (pl.ds(c, 1), slice(None), slice(None))
                )
                pl.store(
                    partial_ref,
                    (pl.ds(c, 1), slice(None), slice(None)),
                    cur + rs_recv_ref[s - 1][None],
                )
            rdma = pltpu.make_async_remote_copy(
                src_ref=partial_ref.at[c],
                dst_ref=rs_recv_ref.at[s],
                send_sem=rs_send_sems.at[s],
                recv_sem=rs_recv_sems.at[s],
                device_id=(right,),
                device_id_type=pl.DeviceIdType.MESH,
            )
            rdma.start()
            rdma.wait()

        mine = pl.load(partial_ref, (pl.ds(my, 1), slice(None), slice(None)))
        out_ref[:, :, :] = mine + rs_recv_ref[N_DEV - 2][None]

    return pl.pallas_call(
        body,
        out_shape=jax.ShapeDtypeStruct((1, SQ_SHARD, D), jnp.float32),
        in_specs=[
            pl.BlockSpec(memory_space=pltpu.VMEM),
            pl.BlockSpec(memory_space=pltpu.VMEM),
            pl.BlockSpec(memory_space=pltpu.VMEM),
            pl.BlockSpec(memory_space=pltpu.ANY),
            pl.BlockSpec(memory_space=pltpu.ANY),
        ],
        out_specs=pl.BlockSpec(memory_space=pltpu.VMEM),
        scratch_shapes=[
            pltpu.VMEM((N_DEV, SQ_SHARD, D), jnp.float32),
            pltpu.VMEM((HQ_LOCAL, KV_CHUNK, DH), jnp.float32),
            pltpu.VMEM((HQ_LOCAL, KV_CHUNK, DH), jnp.float32),
            pltpu.VMEM((N_DEV, SQ_SHARD, D), jnp.float32),
            pltpu.VMEM((N_DEV - 1, SQ_SHARD, D), jnp.float32),
            pltpu.SemaphoreType.DMA((N_DEV - 1,)),
            pltpu.SemaphoreType.DMA((N_DEV - 1,)),
            pltpu.SemaphoreType.DMA((N_DEV - 1,)),
            pltpu.SemaphoreType.DMA((N_DEV - 1,)),
            pltpu.SemaphoreType.DMA((2, HQ_LOCAL)),
        ],
        compiler_params=pltpu.CompilerParams(collective_id=0),
    )(x, Wq, Wo, K_ext, V_ext)


# baseline (device time: 184271 ns/iter reference)
import jax
import jax.numpy as jnp
from jax import lax
from jax.experimental import pallas as pl
from jax.experimental.pallas import tpu as pltpu

N_DEV = 4
SQ_SHARD = 256
SQ = 1024
D = 1024
HQ_LOCAL = 8
DH = 128
SKV = 4096
KV_CHUNK = 1024
N_KV_CHUNKS = SKV // KV_CHUNK
SCALE = 0.08838834764831843


def kernel(x, Wq, Wo, K_ext, V_ext):
    def body(
        x_ref, wq_ref, wo_ref, k_ref, v_ref, out_ref,
        xg_ref, q_ref, attn_ref, kbuf_ref, vbuf_ref, partial_ref, rs_recv_ref,
        ag_send_sems, ag_recv_sems, rs_send_sems, rs_recv_sems, dma_sems,
    ):
        my = lax.axis_index("i")
        left = lax.rem(my + N_DEV - 1, N_DEV)
        right = lax.rem(my + 1, N_DEV)

        barrier_sem = pltpu.get_barrier_semaphore()
        for nbr in (left, right):
            pl.semaphore_signal(
                barrier_sem, inc=1,
                device_id=(nbr,), device_id_type=pl.DeviceIdType.MESH,
            )
        pl.semaphore_wait(barrier_sem, 2)

        xg_ref[pl.ds(my, 1), :, :] = x_ref[:, :, :]
        for h in range(N_DEV - 1):
            src_slot = lax.rem(my - h + N_DEV, N_DEV)
            rdma = pltpu.make_async_remote_copy(
                src_ref=xg_ref.at[src_slot],
                dst_ref=xg_ref.at[src_slot],
                send_sem=ag_send_sems.at[h],
                recv_sem=ag_recv_sems.at[h],
                device_id=(right,),
                device_id_type=pl.DeviceIdType.MESH,
            )
            rdma.start()
            rdma.wait()

        xg = xg_ref[:, :, :].reshape(SQ, D)
        q_ref[:, :] = jnp.dot(
            xg, wq_ref[:, :], preferred_element_type=jnp.float32
        )

        h0 = my * HQ_LOCAL
        pairs = [(h, c) for h in range(HQ_LOCAL) for c in range(N_KV_CHUNKS)]

        def start_fetch(idx):
            h, c = pairs[idx]
            slot = idx % 2
            ck = pltpu.make_async_copy(
                k_ref.at[0, pl.ds(c * KV_CHUNK, KV_CHUNK), h0 + h, :],
                kbuf_ref.at[slot],
                dma_sems.at[0, slot],
            )
            cv = pltpu.make_async_copy(
                v_ref.at[0, pl.ds(c * KV_CHUNK, KV_CHUNK), h0 + h, :],
                vbuf_ref.at[slot],
                dma_sems.at[1, slot],
            )
            ck.start()
            cv.start()
            return ck, cv

        inflight = {0: start_fetch(0)}
        m = l = acc = None
        for idx, (h, c) in enumerate(pairs):
            if idx + 1 < len(pairs):
                inflight[idx + 1] = start_fetch(idx + 1)
            ck, cv = inflight.pop(idx)
            ck.wait()
            cv.wait()
            slot = idx % 2
            qh = q_ref[:, h * DH:(h + 1) * DH]
            kh = kbuf_ref[slot]
            vh = vbuf_ref[slot]
            s = lax.dot_general(
                qh, kh, (((1,), (1,)), ((), ())),
                preferred_element_type=jnp.float32,
            ) * SCALE
            mj = jnp.max(s, axis=1, keepdims=True)
            p = jnp.exp(s - mj)
            lj = jnp.sum(p, axis=1, keepdims=True)
            pv = lax.dot_general(
                p, vh, (((1,), (0,)), ((), ())),
                preferred_element_type=jnp.float32,
            )
            if c == 0:
                m, l, acc = mj, lj, pv
            else:
                m_new = jnp.maximum(m, mj)
                a_old = jnp.exp(m - m_new)
                a_new = jnp.exp(mj - m_new)
                l = l * a_old + lj * a_new
                acc = acc * a_old + pv * a_new
                m = m_new
            if c == N_KV_CHUNKS - 1:
                attn_ref[:, h * DH:(h + 1) * DH] = acc / l

        partial = jnp.dot(
            attn_ref[:, :], wo_ref[:, :], preferred_element_type=jnp.float32
        )
        partial_ref[:, :, :] = partial.reshape(N_DEV, SQ_SHARD, D)

        for s in range(N_DEV - 1):
            c = lax.rem(my - 1 - s + 2 * N_DEV, N_DEV)
            if s > 0:
                cur = partial_ref[pl.ds(c, 1), :, :]
                partial_ref[pl.ds(c, 1), :, :] = cur + rs_recv_ref[s - 1][None]
            rdma = pltpu.make_async_remote_copy(
                src_ref=partial_ref.at[c],
                dst_ref=rs_recv_ref.at[s],
                send_sem=rs_send_sems.at[s],
                recv_sem=rs_recv_sems.at[s],
                device_id=(right,),
                device_id_type=pl.DeviceIdType.MESH,
            )
            rdma.start()
            rdma.wait()

        mine = partial_ref[pl.ds(my, 1), :, :]
        out_ref[:, :, :] = mine + rs_recv_ref[N_DEV - 2][None]

    return pl.pallas_call(
        body,
        out_shape=jax.ShapeDtypeStruct((1, SQ_SHARD, D), jnp.float32),
        in_specs=[
            pl.BlockSpec(memory_space=pltpu.VMEM),
            pl.BlockSpec(memory_space=pltpu.VMEM),
            pl.BlockSpec(memory_space=pltpu.VMEM),
            pl.BlockSpec(memory_space=pl.ANY),
            pl.BlockSpec(memory_space=pl.ANY),
        ],
        out_specs=pl.BlockSpec(memory_space=pltpu.VMEM),
        scratch_shapes=[
            pltpu.VMEM((N_DEV, SQ_SHARD, D), jnp.float32),
            pltpu.VMEM((SQ, D), jnp.float32),
            pltpu.VMEM((SQ, D), jnp.float32),
            pltpu.VMEM((2, KV_CHUNK, DH), jnp.float32),
            pltpu.VMEM((2, KV_CHUNK, DH), jnp.float32),
            pltpu.VMEM((N_DEV, SQ_SHARD, D), jnp.float32),
            pltpu.VMEM((N_DEV - 1, SQ_SHARD, D), jnp.float32),
            pltpu.SemaphoreType.DMA((N_DEV - 1,)),
            pltpu.SemaphoreType.DMA((N_DEV - 1,)),
            pltpu.SemaphoreType.DMA((N_DEV - 1,)),
            pltpu.SemaphoreType.DMA((N_DEV - 1,)),
            pltpu.SemaphoreType.DMA((2, 2)),
        ],
        compiler_params=pltpu.CompilerParams(
            collective_id=0,
            vmem_limit_bytes=100 * 1024 * 1024,
        ),
    )(x, Wq, Wo, K_ext, V_ext)


# device time: 100363 ns/iter; 1.8360x vs baseline; 1.8360x over previous
import jax
import jax.numpy as jnp
from jax import lax
from jax.experimental import pallas as pl
from jax.experimental.pallas import tpu as pltpu

N_DEV = 4
SQ_SHARD = 256
SQ = 1024
D = 1024
HQ_LOCAL = 8
DH = 128
SKV = 4096
SCALE = 0.08838834764831843


def kernel(x, Wq, Wo, K_ext, V_ext):
    def body(
        x_ref, wq_ref, wo_ref, k_ref, v_ref, out_ref,
        xg_ref, kb_ref, vb_ref, stage_ref, partial_ref, rs_recv_ref,
        ag_send_sems, ag_recv_sems, rs_send_sems, rs_recv_sems, stage_sems,
    ):
        my = lax.axis_index("i")
        left = lax.rem(my + N_DEV - 1, N_DEV)
        right = lax.rem(my + 1, N_DEV)

        barrier_sem = pltpu.get_barrier_semaphore()
        for nbr in (left, right):
            pl.semaphore_signal(
                barrier_sem, inc=1,
                device_id=(nbr,), device_id_type=pl.DeviceIdType.MESH,
            )
        pl.semaphore_wait(barrier_sem, 2)

        xg_ref[pl.ds(my, 1), :, :] = x_ref[:, :, :]

        def ag_copy(h, slot):
            return pltpu.make_async_remote_copy(
                src_ref=xg_ref.at[slot],
                dst_ref=xg_ref.at[slot],
                send_sem=ag_send_sems.at[h],
                recv_sem=ag_recv_sems.at[h],
                device_id=(right,),
                device_id_type=pl.DeviceIdType.MESH,
            )

        def rs_copy(s, slot):
            return pltpu.make_async_remote_copy(
                src_ref=partial_ref.at[slot],
                dst_ref=rs_recv_ref.at[s],
                send_sem=rs_send_sems.at[s],
                recv_sem=rs_recv_sems.at[s],
                device_id=(right,),
                device_id_type=pl.DeviceIdType.MESH,
            )

        pending_sends = []

        ag0 = ag_copy(0, my)
        ag0.start()
        pending_sends.append(ag0)

        h0 = my * HQ_LOCAL

        def start_head_fetch(h):
            ks, vs = (2 * h) % 4, (2 * h + 1) % 4
            ck = pltpu.make_async_copy(
                k_ref.at[0, :, h0 + h, :], stage_ref.at[ks], stage_sems.at[ks]
            )
            cv = pltpu.make_async_copy(
                v_ref.at[0, :, h0 + h, :], stage_ref.at[vs], stage_sems.at[vs]
            )
            ck.start()
            cv.start()
            return ck, cv

        fetches = {0: start_head_fetch(0), 1: start_head_fetch(1)}

        wqb = wq_ref[:, :].astype(jnp.bfloat16)
        wob = wo_ref[:, :].astype(jnp.bfloat16)
        rs_rdmas = [None] * (N_DEV - 1)

        for j in range(N_DEV):
            sc = lax.rem(my - j + N_DEV, N_DEV)
            if j > 0:
                ag_copy(j - 1, sc).wait_recv()
                if j < N_DEV - 1:
                    fwd = ag_copy(j, sc)
                    fwd.start()
                    pending_sends.append(fwd)

            xc = xg_ref[pl.ds(sc, 1), :, :].reshape(SQ_SHARD, D)
            qc = jnp.dot(
                xc.astype(jnp.bfloat16), wqb,
                preferred_element_type=jnp.float32,
            ).astype(jnp.bfloat16)
            head_outs = []
            for h in range(HQ_LOCAL):
                if j == 0:
                    ck, cv = fetches.pop(h)
                    ck.wait()
                    cv.wait()
                    kb_ref[h] = stage_ref[(2 * h) % 4].astype(jnp.bfloat16)
                    vb_ref[h] = stage_ref[(2 * h + 1) % 4].astype(jnp.bfloat16)
                    if h + 2 < HQ_LOCAL:
                        fetches[h + 2] = start_head_fetch(h + 2)
                qh = qc[:, h * DH:(h + 1) * DH]
                s_blk = lax.dot_general(
                    qh, kb_ref[h], (((1,), (1,)), ((), ())),
                    preferred_element_type=jnp.float32,
                ) * SCALE
                mh = jnp.max(s_blk, axis=1, keepdims=True)
                p = jnp.exp(s_blk - mh)
                lh = jnp.sum(p, axis=1, keepdims=True)
                pv = lax.dot_general(
                    p.astype(jnp.bfloat16), vb_ref[h], (((1,), (0,)), ((), ())),
                    preferred_element_type=jnp.float32,
                )
                head_outs.append((pv / lh).astype(jnp.bfloat16))
            attn_c = jnp.concatenate(head_outs, axis=1)
            pc = jnp.dot(attn_c, wob, preferred_element_type=jnp.float32)

            if j == 0:
                partial_ref[pl.ds(sc, 1), :, :] = pc[None]
            else:
                s = j - 1
                if s > 0:
                    rs_rdmas[s - 1].wait_recv()
                    pc = pc + rs_recv_ref[s - 1]
                partial_ref[pl.ds(sc, 1), :, :] = pc[None]
                rs = rs_copy(s, sc)
                rs.start()
                rs_rdmas[s] = rs
                pending_sends.append(rs)

        rs_rdmas[N_DEV - 2].wait_recv()
        mine = partial_ref[pl.ds(my, 1), :, :]
        out_ref[:, :, :] = mine + rs_recv_ref[N_DEV - 2][None]

        for r in pending_sends:
            r.wait_send()

    return pl.pallas_call(
        body,
        out_shape=jax.ShapeDtypeStruct((1, SQ_SHARD, D), jnp.float32),
        in_specs=[
            pl.BlockSpec(memory_space=pltpu.VMEM),
            pl.BlockSpec(memory_space=pltpu.VMEM),
            pl.BlockSpec(memory_space=pltpu.VMEM),
            pl.BlockSpec(memory_space=pl.ANY),
            pl.BlockSpec(memory_space=pl.ANY),
        ],
        out_specs=pl.BlockSpec(memory_space=pltpu.VMEM),
        scratch_shapes=[
            pltpu.VMEM((N_DEV, SQ_SHARD, D), jnp.float32),
            pltpu.VMEM((HQ_LOCAL, SKV, DH), jnp.bfloat16),
            pltpu.VMEM((HQ_LOCAL, SKV, DH), jnp.bfloat16),
            pltpu.VMEM((4, SKV, DH), jnp.float32),
            pltpu.VMEM((N_DEV, SQ_SHARD, D), jnp.float32),
            pltpu.VMEM((N_DEV - 1, SQ_SHARD, D), jnp.float32),
            pltpu.SemaphoreType.DMA((N_DEV - 1,)),
            pltpu.SemaphoreType.DMA((N_DEV - 1,)),
            pltpu.SemaphoreType.DMA((N_DEV - 1,)),
            pltpu.SemaphoreType.DMA((N_DEV - 1,)),
            pltpu.SemaphoreType.DMA((4,)),
        ],
        compiler_params=pltpu.CompilerParams(
            collective_id=0,
            vmem_limit_bytes=100 * 1024 * 1024,
        ),
    )(x, Wq, Wo, K_ext, V_ext)


# device time: 87146 ns/iter; 2.1145x vs baseline; 1.1517x over previous
import jax
import jax.numpy as jnp
from jax import lax
from jax.experimental import pallas as pl
from jax.experimental.pallas import tpu as pltpu

N_DEV = 4
SQ_SHARD = 256
SQ = 1024
D = 1024
HQ_LOCAL = 8
DH = 128
SKV = 4096
SCALE = 0.08838834764831843


def kernel(x, Wq, Wo, K_ext, V_ext):
    def body(
        x_ref, wq_ref, wo_ref, k_ref, v_ref, out_ref,
        xg_ref, kb_ref, vb_ref, stage_ref, partial_ref, rs_recv_ref,
        ag_send_sems, ag_recv_sems, rs_send_sems, rs_recv_sems, stage_sems,
    ):
        my = lax.axis_index("i")
        left = lax.rem(my + N_DEV - 1, N_DEV)
        right = lax.rem(my + 1, N_DEV)

        xg_ref[pl.ds(my, 1), :, :] = x_ref[:, :, :]

        def ag_copy(h, slot):
            return pltpu.make_async_remote_copy(
                src_ref=xg_ref.at[slot],
                dst_ref=xg_ref.at[slot],
                send_sem=ag_send_sems.at[h],
                recv_sem=ag_recv_sems.at[h],
                device_id=(right,),
                device_id_type=pl.DeviceIdType.MESH,
            )

        def rs_copy(s, slot):
            return pltpu.make_async_remote_copy(
                src_ref=partial_ref.at[slot],
                dst_ref=rs_recv_ref.at[s],
                send_sem=rs_send_sems.at[s],
                recv_sem=rs_recv_sems.at[s],
                device_id=(right,),
                device_id_type=pl.DeviceIdType.MESH,
            )

        pending_sends = []

        h0 = my * HQ_LOCAL

        def start_head_fetch(h):
            ks, vs = (2 * h) % 4, (2 * h + 1) % 4
            ck = pltpu.make_async_copy(
                k_ref.at[0, :, h0 + h, :], stage_ref.at[ks], stage_sems.at[ks]
            )
            cv = pltpu.make_async_copy(
                v_ref.at[0, :, h0 + h, :], stage_ref.at[vs], stage_sems.at[vs]
            )
            ck.start()
            cv.start()
            return ck, cv

        fetches = {0: start_head_fetch(0), 1: start_head_fetch(1)}

        barrier_sem = pltpu.get_barrier_semaphore()
        for nbr in (left, right):
            pl.semaphore_signal(
                barrier_sem, inc=1,
                device_id=(nbr,), device_id_type=pl.DeviceIdType.MESH,
            )
        pl.semaphore_wait(barrier_sem, 2)

        ag0 = ag_copy(0, my)
        ag0.start()
        pending_sends.append(ag0)

        wqb = wq_ref[:, :].astype(jnp.bfloat16)
        wob = wo_ref[:, :].astype(jnp.bfloat16)
        rs_rdmas = [None] * (N_DEV - 1)

        for j in range(N_DEV):
            sc = lax.rem(my - j + N_DEV, N_DEV)
            if j > 0:
                ag_copy(j - 1, sc).wait_recv()
                if j < N_DEV - 1:
                    fwd = ag_copy(j, sc)
                    fwd.start()
                    pending_sends.append(fwd)

            xc = xg_ref[pl.ds(sc, 1), :, :].reshape(SQ_SHARD, D)
            qc = (jnp.dot(
                xc.astype(jnp.bfloat16), wqb,
                preferred_element_type=jnp.float32,
            ) * SCALE).astype(jnp.bfloat16)
            head_outs = []
            for h in range(HQ_LOCAL):
                if j == 0:
                    ck, cv = fetches.pop(h)
                    ck.wait()
                    cv.wait()
                    kb_ref[h] = stage_ref[(2 * h) % 4].astype(jnp.bfloat16)
                    vb_ref[h] = stage_ref[(2 * h + 1) % 4].astype(jnp.bfloat16)
                    if h + 2 < HQ_LOCAL:
                        fetches[h + 2] = start_head_fetch(h + 2)
                qh = qc[:, h * DH:(h + 1) * DH]
                s_blk = lax.dot_general(
                    qh, kb_ref[h], (((1,), (1,)), ((), ())),
                    preferred_element_type=jnp.float32,
                )
                p = jnp.exp(s_blk)
                lh = jnp.sum(p, axis=1, keepdims=True)
                pv = lax.dot_general(
                    p.astype(jnp.bfloat16), vb_ref[h], (((1,), (0,)), ((), ())),
                    preferred_element_type=jnp.float32,
                )
                head_outs.append((pv / lh).astype(jnp.bfloat16))
            attn_c = jnp.concatenate(head_outs, axis=1)
            pc = jnp.dot(attn_c, wob, preferred_element_type=jnp.float32)

            if j == 0:
                partial_ref[pl.ds(sc, 1), :, :] = pc[None]
            else:
                s = j - 1
                if s > 0:
                    rs_rdmas[s - 1].wait_recv()
                    pc = pc + rs_recv_ref[s - 1]
                partial_ref[pl.ds(sc, 1), :, :] = pc[None]
                rs = rs_copy(s, sc)
                rs.start()
                rs_rdmas[s] = rs
                pending_sends.append(rs)

        rs_rdmas[N_DEV - 2].wait_recv()
        mine = partial_ref[pl.ds(my, 1), :, :]
        out_ref[:, :, :] = mine + rs_recv_ref[N_DEV - 2][None]

        for r in pending_sends:
            r.wait_send()

    return pl.pallas_call(
        body,
        out_shape=jax.ShapeDtypeStruct((1, SQ_SHARD, D), jnp.float32),
        in_specs=[
            pl.BlockSpec(memory_space=pltpu.VMEM),
            pl.BlockSpec(memory_space=pltpu.VMEM),
            pl.BlockSpec(memory_space=pltpu.VMEM),
            pl.BlockSpec(memory_space=pl.ANY),
            pl.BlockSpec(memory_space=pl.ANY),
        ],
        out_specs=pl.BlockSpec(memory_space=pltpu.VMEM),
        scratch_shapes=[
            pltpu.VMEM((N_DEV, SQ_SHARD, D), jnp.float32),
            pltpu.VMEM((HQ_LOCAL, SKV, DH), jnp.bfloat16),
            pltpu.VMEM((HQ_LOCAL, SKV, DH), jnp.bfloat16),
            pltpu.VMEM((4, SKV, DH), jnp.float32),
            pltpu.VMEM((N_DEV, SQ_SHARD, D), jnp.float32),
            pltpu.VMEM((N_DEV - 1, SQ_SHARD, D), jnp.float32),
            pltpu.SemaphoreType.DMA((N_DEV - 1,)),
            pltpu.SemaphoreType.DMA((N_DEV - 1,)),
            pltpu.SemaphoreType.DMA((N_DEV - 1,)),
            pltpu.SemaphoreType.DMA((N_DEV - 1,)),
            pltpu.SemaphoreType.DMA((4,)),
        ],
        compiler_params=pltpu.CompilerParams(
            collective_id=0,
            vmem_limit_bytes=100 * 1024 * 1024,
        ),
    )(x, Wq, Wo, K_ext, V_ext)


# device time: 83319 ns/iter; 2.2116x vs baseline; 1.0459x over previous
import jax
import jax.numpy as jnp
from jax import lax
from jax.experimental import pallas as pl
from jax.experimental.pallas import tpu as pltpu

N_DEV = 4
SQ_SHARD = 256
SQ = 1024
D = 1024
HQ_LOCAL = 8
DH = 128
SKV = 4096
SCALE = 0.08838834764831843


def kernel(x, Wq, Wo, K_ext, V_ext):
    def body(
        x_ref, wq_ref, wo_ref, k_ref, v_ref, out_ref,
        xg_ref, kb_ref, vb_ref, stage_ref, partial_ref, rs_recv_ref,
        ag_send_sems, ag_recv_sems, rs_send_sems, rs_recv_sems, stage_sems,
    ):
        my = lax.axis_index("i")
        left = lax.rem(my + N_DEV - 1, N_DEV)
        right = lax.rem(my + 1, N_DEV)

        xg_ref[pl.ds(my, 1), :, :] = x_ref[:, :, :].astype(jnp.bfloat16)

        def ag_copy(h, slot):
            return pltpu.make_async_remote_copy(
                src_ref=xg_ref.at[slot],
                dst_ref=xg_ref.at[slot],
                send_sem=ag_send_sems.at[h],
                recv_sem=ag_recv_sems.at[h],
                device_id=(right,),
                device_id_type=pl.DeviceIdType.MESH,
            )

        def rs_copy(s, slot):
            return pltpu.make_async_remote_copy(
                src_ref=partial_ref.at[slot],
                dst_ref=rs_recv_ref.at[s],
                send_sem=rs_send_sems.at[s],
                recv_sem=rs_recv_sems.at[s],
                device_id=(right,),
                device_id_type=pl.DeviceIdType.MESH,
            )

        pending_sends = []

        h0 = my * HQ_LOCAL

        def start_head_fetch(h):
            ks, vs = (2 * h) % 4, (2 * h + 1) % 4
            ck = pltpu.make_async_copy(
                k_ref.at[0, :, h0 + h, :], stage_ref.at[ks], stage_sems.at[ks]
            )
            cv = pltpu.make_async_copy(
                v_ref.at[0, :, h0 + h, :], stage_ref.at[vs], stage_sems.at[vs]
            )
            ck.start()
            cv.start()
            return ck, cv

        fetches = {0: start_head_fetch(0), 1: start_head_fetch(1)}

        barrier_sem = pltpu.get_barrier_semaphore()
        for nbr in (left, right):
            pl.semaphore_signal(
                barrier_sem, inc=1,
                device_id=(nbr,), device_id_type=pl.DeviceIdType.MESH,
            )
        pl.semaphore_wait(barrier_sem, 2)

        ag0 = ag_copy(0, my)
        ag0.start()
        pending_sends.append(ag0)

        wqb = wq_ref[:, :].astype(jnp.bfloat16)
        wob = wo_ref[:, :].astype(jnp.bfloat16)
        rs_rdmas = [None] * (N_DEV - 1)

        for j in range(N_DEV):
            sc = lax.rem(my - j + N_DEV, N_DEV)
            if j > 0:
                ag_copy(j - 1, sc).wait_recv()
                if j < N_DEV - 1:
                    fwd = ag_copy(j, sc)
                    fwd.start()
                    pending_sends.append(fwd)

            xc = xg_ref[pl.ds(sc, 1), :, :].reshape(SQ_SHARD, D)
            qc = (jnp.dot(
                xc, wqb, preferred_element_type=jnp.float32,
            ) * SCALE).astype(jnp.bfloat16)
            head_outs = []
            for h in range(HQ_LOCAL):
                if j == 0:
                    ck, cv = fetches.pop(h)
                    ck.wait()
                    cv.wait()
                    kb_ref[h] = stage_ref[(2 * h) % 4].astype(jnp.bfloat16)
                    vb_ref[h] = stage_ref[(2 * h + 1) % 4].astype(jnp.bfloat16)
                    if h + 2 < HQ_LOCAL:
                        fetches[h + 2] = start_head_fetch(h + 2)
                qh = qc[:, h * DH:(h + 1) * DH]
                s_blk = lax.dot_general(
                    qh, kb_ref[h], (((1,), (1,)), ((), ())),
                    preferred_element_type=jnp.float32,
                )
                p = jnp.exp(s_blk).astype(jnp.bfloat16)
                lh = jnp.sum(p, axis=1, keepdims=True, dtype=jnp.float32)
                pv = lax.dot_general(
                    p, vb_ref[h], (((1,), (0,)), ((), ())),
                    preferred_element_type=jnp.float32,
                )
                head_outs.append((pv / lh).astype(jnp.bfloat16))
            attn_c = jnp.concatenate(head_outs, axis=1)
            pc = jnp.dot(attn_c, wob, preferred_element_type=jnp.float32)

            if j == 0:
                partial_ref[pl.ds(sc, 1), :, :] = pc[None]
            else:
                s = j - 1
                if s > 0:
                    rs_rdmas[s - 1].wait_recv()
                    pc = pc + rs_recv_ref[s - 1]
                partial_ref[pl.ds(sc, 1), :, :] = pc[None]
                rs = rs_copy(s, sc)
                rs.start()
                rs_rdmas[s] = rs
                pending_sends.append(rs)

        rs_rdmas[N_DEV - 2].wait_recv()
        mine = partial_ref[pl.ds(my, 1), :, :]
        out_ref[:, :, :] = mine + rs_recv_ref[N_DEV - 2][None]

        for r in pending_sends:
            r.wait_send()

    return pl.pallas_call(
        body,
        out_shape=jax.ShapeDtypeStruct((1, SQ_SHARD, D), jnp.float32),
        in_specs=[
            pl.BlockSpec(memory_space=pltpu.VMEM),
            pl.BlockSpec(memory_space=pltpu.VMEM),
            pl.BlockSpec(memory_space=pltpu.VMEM),
            pl.BlockSpec(memory_space=pl.ANY),
            pl.BlockSpec(memory_space=pl.ANY),
        ],
        out_specs=pl.BlockSpec(memory_space=pltpu.VMEM),
        scratch_shapes=[
            pltpu.VMEM((N_DEV, SQ_SHARD, D), jnp.bfloat16),
            pltpu.VMEM((HQ_LOCAL, SKV, DH), jnp.bfloat16),
            pltpu.VMEM((HQ_LOCAL, SKV, DH), jnp.bfloat16),
            pltpu.VMEM((4, SKV, DH), jnp.float32),
            pltpu.VMEM((N_DEV, SQ_SHARD, D), jnp.float32),
            pltpu.VMEM((N_DEV - 1, SQ_SHARD, D), jnp.float32),
            pltpu.SemaphoreType.DMA((N_DEV - 1,)),
            pltpu.SemaphoreType.DMA((N_DEV - 1,)),
            pltpu.SemaphoreType.DMA((N_DEV - 1,)),
            pltpu.SemaphoreType.DMA((N_DEV - 1,)),
            pltpu.SemaphoreType.DMA((4,)),
        ],
        compiler_params=pltpu.CompilerParams(
            collective_id=0,
            vmem_limit_bytes=100 * 1024 * 1024,
        ),
    )(x, Wq, Wo, K_ext, V_ext)


# device time: 71810 ns/iter; 2.5661x vs baseline; 1.1603x over previous
import jax
import jax.numpy as jnp
from jax import lax
from jax.experimental import pallas as pl
from jax.experimental.pallas import tpu as pltpu

N_DEV = 4
SQ_SHARD = 256
SQ = 1024
D = 1024
HQ_LOCAL = 8
DH = 128
SKV = 4096
SCALE = 0.08838834764831843


def kernel(x, Wq, Wo, K_ext, V_ext):
    def body(
        x_ref, wq_ref, wo_ref, k_ref, v_ref, out_ref,
        xg_ref, kb_ref, vb_ref, stage_ref, pmine_ref, rs_send_ref, rs_recv_ref,
        ag_send_sems, ag_recv_sems, rs_send_sems, rs_recv_sems, stage_sems,
    ):
        my = lax.axis_index("i")
        left = lax.rem(my + N_DEV - 1, N_DEV)
        right = lax.rem(my + 1, N_DEV)

        xg_ref[pl.ds(my, 1), :, :] = x_ref[:, :, :].astype(jnp.bfloat16)

        def ag_copy(h, slot):
            return pltpu.make_async_remote_copy(
                src_ref=xg_ref.at[slot],
                dst_ref=xg_ref.at[slot],
                send_sem=ag_send_sems.at[h],
                recv_sem=ag_recv_sems.at[h],
                device_id=(right,),
                device_id_type=pl.DeviceIdType.MESH,
            )

        def rs_copy(s):
            return pltpu.make_async_remote_copy(
                src_ref=rs_send_ref.at[s],
                dst_ref=rs_recv_ref.at[s],
                send_sem=rs_send_sems.at[s],
                recv_sem=rs_recv_sems.at[s],
                device_id=(right,),
                device_id_type=pl.DeviceIdType.MESH,
            )

        pending_sends = []

        h0 = my * HQ_LOCAL

        def start_head_fetch(h):
            ks, vs = (2 * h) % 4, (2 * h + 1) % 4
            ck = pltpu.make_async_copy(
                k_ref.at[0, :, h0 + h, :], stage_ref.at[ks], stage_sems.at[ks]
            )
            cv = pltpu.make_async_copy(
                v_ref.at[0, :, h0 + h, :], stage_ref.at[vs], stage_sems.at[vs]
            )
            ck.start()
            cv.start()
            return ck, cv

        fetches = {0: start_head_fetch(0), 1: start_head_fetch(1)}

        barrier_sem = pltpu.get_barrier_semaphore()
        for nbr in (left, right):
            pl.semaphore_signal(
                barrier_sem, inc=1,
                device_id=(nbr,), device_id_type=pl.DeviceIdType.MESH,
            )
        pl.semaphore_wait(barrier_sem, 2)

        ag0 = ag_copy(0, my)
        ag0.start()
        pending_sends.append(ag0)

        wqb = wq_ref[:, :].astype(jnp.bfloat16)
        wob = wo_ref[:, :].astype(jnp.bfloat16)
        rs_rdmas = [None] * (N_DEV - 1)

        for j in range(N_DEV):
            sc = lax.rem(my - j + N_DEV, N_DEV)
            if j > 0:
                ag_copy(j - 1, sc).wait_recv()
                if j < N_DEV - 1:
                    fwd = ag_copy(j, sc)
                    fwd.start()
                    pending_sends.append(fwd)

            xc = xg_ref[pl.ds(sc, 1), :, :].reshape(SQ_SHARD, D)
            qc = (jnp.dot(
                xc, wqb, preferred_element_type=jnp.float32,
            ) * SCALE).astype(jnp.bfloat16)
            head_outs = []
            for h in range(HQ_LOCAL):
                if j == 0:
                    ck, cv = fetches.pop(h)
                    ck.wait()
                    cv.wait()
                    kb_ref[h] = stage_ref[(2 * h) % 4].astype(jnp.bfloat16)
                    vb_ref[h] = stage_ref[(2 * h + 1) % 4].astype(jnp.bfloat16)
                    if h + 2 < HQ_LOCAL:
                        fetches[h + 2] = start_head_fetch(h + 2)
                qh = qc[:, h * DH:(h + 1) * DH]
                s_blk = lax.dot_general(
                    qh, kb_ref[h], (((1,), (1,)), ((), ())),
                    preferred_element_type=jnp.float32,
                )
                p = jnp.exp(s_blk).astype(jnp.bfloat16)
                lh = jnp.sum(p, axis=1, keepdims=True, dtype=jnp.float32)
                pv = lax.dot_general(
                    p, vb_ref[h], (((1,), (0,)), ((), ())),
                    preferred_element_type=jnp.float32,
                )
                head_outs.append((pv / lh).astype(jnp.bfloat16))
            attn_c = jnp.concatenate(head_outs, axis=1)
            pc = jnp.dot(attn_c, wob, preferred_element_type=jnp.float32)

            if j == 0:
                pmine_ref[:, :] = pc
            else:
                s = j - 1
                if s > 0:
                    rs_rdmas[s - 1].wait_recv()
                    pc = pc + rs_recv_ref[s - 1].astype(jnp.float32)
                rs_send_ref[s] = pc.astype(jnp.bfloat16)
                rs = rs_copy(s)
                rs.start()
                rs_rdmas[s] = rs
                pending_sends.append(rs)

        rs_rdmas[N_DEV - 2].wait_recv()
        final = pmine_ref[:, :] + rs_recv_ref[N_DEV - 2].astype(jnp.float32)
        out_ref[:, :, :] = final[None]

        for r in pending_sends:
            r.wait_send()

    return pl.pallas_call(
        body,
        out_shape=jax.ShapeDtypeStruct((1, SQ_SHARD, D), jnp.float32),
        in_specs=[
            pl.BlockSpec(memory_space=pltpu.VMEM),
            pl.BlockSpec(memory_space=pltpu.VMEM),
            pl.BlockSpec(memory_space=pltpu.VMEM),
            pl.BlockSpec(memory_space=pl.ANY),
            pl.BlockSpec(memory_space=pl.ANY),
        ],
        out_specs=pl.BlockSpec(memory_space=pltpu.VMEM),
        scratch_shapes=[
            pltpu.VMEM((N_DEV, SQ_SHARD, D), jnp.bfloat16),
            pltpu.VMEM((HQ_LOCAL, SKV, DH), jnp.bfloat16),
            pltpu.VMEM((HQ_LOCAL, SKV, DH), jnp.bfloat16),
            pltpu.VMEM((4, SKV, DH), jnp.float32),
            pltpu.VMEM((SQ_SHARD, D), jnp.float32),
            pltpu.VMEM((N_DEV - 1, SQ_SHARD, D), jnp.bfloat16),
            pltpu.VMEM((N_DEV - 1, SQ_SHARD, D), jnp.bfloat16),
            pltpu.SemaphoreType.DMA((N_DEV - 1,)),
            pltpu.SemaphoreType.DMA((N_DEV - 1,)),
            pltpu.SemaphoreType.DMA((N_DEV - 1,)),
            pltpu.SemaphoreType.DMA((N_DEV - 1,)),
            pltpu.SemaphoreType.DMA((4,)),
        ],
        compiler_params=pltpu.CompilerParams(
            collective_id=0,
            vmem_limit_bytes=100 * 1024 * 1024,
        ),
    )(x, Wq, Wo, K_ext, V_ext)


# device time: 70398 ns/iter; 2.6176x vs baseline; 1.0201x over previous
import jax
import jax.numpy as jnp
from jax import lax
from jax.experimental import pallas as pl
from jax.experimental.pallas import tpu as pltpu

N_DEV = 4
SQ_SHARD = 256
SQ = 1024
D = 1024
HQ_LOCAL = 8
DH = 128
SKV = 4096
SCALE = 0.08838834764831843


def kernel(x, Wq, Wo, K_ext, V_ext):
    def body(
        x_ref, wq_ref, wo_ref, k_ref, v_ref, out_ref,
        xg_ref, kb_ref, vb_ref, stage_ref, pmine_ref, rs_send_ref, rs_recv_ref,
        ag_send_sems, ag_recv_sems, rs_send_sems, rs_recv_sems, stage_sems,
    ):
        my = lax.axis_index("i")
        left = lax.rem(my + N_DEV - 1, N_DEV)
        right = lax.rem(my + 1, N_DEV)

        xg_ref[pl.ds(my, 1), :, :] = x_ref[:, :, :].astype(jnp.bfloat16)

        def ag_copy(h, slot):
            return pltpu.make_async_remote_copy(
                src_ref=xg_ref.at[slot],
                dst_ref=xg_ref.at[slot],
                send_sem=ag_send_sems.at[h],
                recv_sem=ag_recv_sems.at[h],
                device_id=(right,),
                device_id_type=pl.DeviceIdType.MESH,
            )

        def rs_copy(s):
            return pltpu.make_async_remote_copy(
                src_ref=rs_send_ref.at[s],
                dst_ref=rs_recv_ref.at[s],
                send_sem=rs_send_sems.at[s],
                recv_sem=rs_recv_sems.at[s],
                device_id=(right,),
                device_id_type=pl.DeviceIdType.MESH,
            )

        pending_sends = []

        h0 = my * HQ_LOCAL

        def start_head_fetch(h):
            ks, vs = (2 * h) % 8, (2 * h + 1) % 8
            ck = pltpu.make_async_copy(
                k_ref.at[0, :, h0 + h, :], stage_ref.at[ks], stage_sems.at[ks]
            )
            cv = pltpu.make_async_copy(
                v_ref.at[0, :, h0 + h, :], stage_ref.at[vs], stage_sems.at[vs]
            )
            ck.start()
            cv.start()
            return ck, cv

        fetches = {h: start_head_fetch(h) for h in range(4)}

        barrier_sem = pltpu.get_barrier_semaphore()
        for nbr in (left, right):
            pl.semaphore_signal(
                barrier_sem, inc=1,
                device_id=(nbr,), device_id_type=pl.DeviceIdType.MESH,
            )
        pl.semaphore_wait(barrier_sem, 2)

        ag0 = ag_copy(0, my)
        ag0.start()
        pending_sends.append(ag0)

        wqb = wq_ref[:, :].astype(jnp.bfloat16)
        wob = wo_ref[:, :].astype(jnp.bfloat16)
        rs_rdmas = [None] * (N_DEV - 1)

        for j in range(N_DEV):
            sc = lax.rem(my - j + N_DEV, N_DEV)
            if j > 0:
                ag_copy(j - 1, sc).wait_recv()
                if j < N_DEV - 1:
                    fwd = ag_copy(j, sc)
                    fwd.start()
                    pending_sends.append(fwd)

            xc = xg_ref[pl.ds(sc, 1), :, :].reshape(SQ_SHARD, D)
            qc = (jnp.dot(
                xc, wqb, preferred_element_type=jnp.float32,
            ) * SCALE).astype(jnp.bfloat16)
            head_outs = []
            for h in range(HQ_LOCAL):
                if j == 0:
                    ck, cv = fetches.pop(h)
                    ck.wait()
                    cv.wait()
                    kb_ref[h] = stage_ref[(2 * h) % 8].astype(jnp.bfloat16)
                    vb_ref[h] = stage_ref[(2 * h + 1) % 8].astype(jnp.bfloat16)
                    if h + 4 < HQ_LOCAL:
                        fetches[h + 4] = start_head_fetch(h + 4)
                qh = qc[:, h * DH:(h + 1) * DH]
                s_blk = lax.dot_general(
                    qh, kb_ref[h], (((1,), (1,)), ((), ())),
                    preferred_element_type=jnp.float32,
                )
                p = jnp.exp(s_blk).astype(jnp.bfloat16)
                lh = jnp.sum(p, axis=1, keepdims=True, dtype=jnp.float32)
                pv = lax.dot_general(
                    p, vb_ref[h], (((1,), (0,)), ((), ())),
                    preferred_element_type=jnp.float32,
                )
                head_outs.append((pv / lh).astype(jnp.bfloat16))
            attn_c = jnp.concatenate(head_outs, axis=1)
            pc = jnp.dot(attn_c, wob, preferred_element_type=jnp.float32)

            if j == 0:
                pmine_ref[:, :] = pc
            else:
                s = j - 1
                if s > 0:
                    rs_rdmas[s - 1].wait_recv()
                    pc = pc + rs_recv_ref[s - 1].astype(jnp.float32)
                rs_send_ref[s] = pc.astype(jnp.bfloat16)
                rs = rs_copy(s)
                rs.start()
                rs_rdmas[s] = rs
                pending_sends.append(rs)

        rs_rdmas[N_DEV - 2].wait_recv()
        final = pmine_ref[:, :] + rs_recv_ref[N_DEV - 2].astype(jnp.float32)
        out_ref[:, :, :] = final[None]

        for r in pending_sends:
            r.wait_send()

    return pl.pallas_call(
        body,
        out_shape=jax.ShapeDtypeStruct((1, SQ_SHARD, D), jnp.float32),
        in_specs=[
            pl.BlockSpec(memory_space=pltpu.VMEM),
            pl.BlockSpec(memory_space=pltpu.VMEM),
            pl.BlockSpec(memory_space=pltpu.VMEM),
            pl.BlockSpec(memory_space=pl.ANY),
            pl.BlockSpec(memory_space=pl.ANY),
        ],
        out_specs=pl.BlockSpec(memory_space=pltpu.VMEM),
        scratch_shapes=[
            pltpu.VMEM((N_DEV, SQ_SHARD, D), jnp.bfloat16),
            pltpu.VMEM((HQ_LOCAL, SKV, DH), jnp.bfloat16),
            pltpu.VMEM((HQ_LOCAL, SKV, DH), jnp.bfloat16),
            pltpu.VMEM((8, SKV, DH), jnp.float32),
            pltpu.VMEM((SQ_SHARD, D), jnp.float32),
            pltpu.VMEM((N_DEV - 1, SQ_SHARD, D), jnp.bfloat16),
            pltpu.VMEM((N_DEV - 1, SQ_SHARD, D), jnp.bfloat16),
            pltpu.SemaphoreType.DMA((N_DEV - 1,)),
            pltpu.SemaphoreType.DMA((N_DEV - 1,)),
            pltpu.SemaphoreType.DMA((N_DEV - 1,)),
            pltpu.SemaphoreType.DMA((N_DEV - 1,)),
            pltpu.SemaphoreType.DMA((8,)),
        ],
        compiler_params=pltpu.CompilerParams(
            collective_id=0,
            vmem_limit_bytes=100 * 1024 * 1024,
        ),
    )(x, Wq, Wo, K_ext, V_ext)


# device time: 66506 ns/iter; 2.7707x vs baseline; 1.0585x over previous
import jax
import jax.numpy as jnp
from jax import lax
from jax.experimental import pallas as pl
from jax.experimental.pallas import tpu as pltpu

N_DEV = 4
SQ_SHARD = 256
SQ = 1024
D = 1024
HQ_LOCAL = 8
DH = 128
SKV = 4096
SCALE = 0.08838834764831843
SCALE2 = SCALE * 1.4426950408889634


def kernel(x, Wq, Wo, K_ext, V_ext):
    def body(
        x_ref, wq_ref, wo_ref, k_ref, v_ref, out_ref,
        xg_ref, kb_ref, vb_ref, stage_ref, pmine_ref, rs_send_ref, rs_recv_ref,
        ag_send_sems, ag_recv_sems, rs_send_sems, rs_recv_sems, stage_sems,
    ):
        my = lax.axis_index("i")
        left = lax.rem(my + N_DEV - 1, N_DEV)
        right = lax.rem(my + 1, N_DEV)

        xg_ref[pl.ds(my, 1), :, :] = x_ref[:, :, :].astype(jnp.bfloat16)

        def ag_copy(h, slot):
            return pltpu.make_async_remote_copy(
                src_ref=xg_ref.at[slot],
                dst_ref=xg_ref.at[slot],
                send_sem=ag_send_sems.at[h],
                recv_sem=ag_recv_sems.at[h],
                device_id=(right,),
                device_id_type=pl.DeviceIdType.MESH,
            )

        def rs_copy(s):
            return pltpu.make_async_remote_copy(
                src_ref=rs_send_ref.at[s],
                dst_ref=rs_recv_ref.at[s],
                send_sem=rs_send_sems.at[s],
                recv_sem=rs_recv_sems.at[s],
                device_id=(right,),
                device_id_type=pl.DeviceIdType.MESH,
            )

        pending_sends = []

        h0 = my * HQ_LOCAL

        def start_head_fetch(h):
            ks, vs = (2 * h) % 8, (2 * h + 1) % 8
            ck = pltpu.make_async_copy(
                k_ref.at[0, :, h0 + h, :], stage_ref.at[ks], stage_sems.at[ks]
            )
            cv = pltpu.make_async_copy(
                v_ref.at[0, :, h0 + h, :], stage_ref.at[vs], stage_sems.at[vs]
            )
            ck.start()
            cv.start()
            return ck, cv

        fetches = {h: start_head_fetch(h) for h in range(4)}

        barrier_sem = pltpu.get_barrier_semaphore()
        for nbr in (left, right):
            pl.semaphore_signal(
                barrier_sem, inc=1,
                device_id=(nbr,), device_id_type=pl.DeviceIdType.MESH,
            )
        pl.semaphore_wait(barrier_sem, 2)

        ag0 = ag_copy(0, my)
        ag0.start()
        pending_sends.append(ag0)

        wqb = wq_ref[:, :].astype(jnp.bfloat16)
        wob = wo_ref[:, :].astype(jnp.bfloat16)
        rs_rdmas = [None] * (N_DEV - 1)

        for j in range(N_DEV):
            sc = lax.rem(my - j + N_DEV, N_DEV)
            if j > 0:
                ag_copy(j - 1, sc).wait_recv()
                if j < N_DEV - 1:
                    fwd = ag_copy(j, sc)
                    fwd.start()
                    pending_sends.append(fwd)

            xc = xg_ref[pl.ds(sc, 1), :, :].reshape(SQ_SHARD, D)
            qc = (jnp.dot(
                xc, wqb, preferred_element_type=jnp.float32,
            ) * SCALE2).astype(jnp.bfloat16)
            head_outs = []
            for h in range(HQ_LOCAL):
                if j == 0:
                    ck, cv = fetches.pop(h)
                    ck.wait()
                    cv.wait()
                    kb_ref[h] = stage_ref[(2 * h) % 8].astype(jnp.bfloat16)
                    vb_ref[h] = stage_ref[(2 * h + 1) % 8].astype(jnp.bfloat16)
                    if h + 4 < HQ_LOCAL:
                        fetches[h + 4] = start_head_fetch(h + 4)
                qh = qc[:, h * DH:(h + 1) * DH]
                s_blk = lax.dot_general(
                    qh, kb_ref[h], (((1,), (1,)), ((), ())),
                    preferred_element_type=jnp.float32,
                )
                p = jnp.exp2(s_blk).astype(jnp.bfloat16)
                lh = jnp.sum(p, axis=1, keepdims=True, dtype=jnp.float32)
                pv = lax.dot_general(
                    p, vb_ref[h], (((1,), (0,)), ((), ())),
                    preferred_element_type=jnp.float32,
                )
                head_outs.append((pv / lh).astype(jnp.bfloat16))
            attn_c = jnp.concatenate(head_outs, axis=1)
            pc = jnp.dot(attn_c, wob, preferred_element_type=jnp.float32)

            if j == 0:
                pmine_ref[:, :] = pc
            else:
                s = j - 1
                if s > 0:
                    rs_rdmas[s - 1].wait_recv()
                    pc = pc + rs_recv_ref[s - 1].astype(jnp.float32)
                rs_send_ref[s] = pc.astype(jnp.bfloat16)
                rs = rs_copy(s)
                rs.start()
                rs_rdmas[s] = rs
                pending_sends.append(rs)

        rs_rdmas[N_DEV - 2].wait_recv()
        final = pmine_ref[:, :] + rs_recv_ref[N_DEV - 2].astype(jnp.float32)
        out_ref[:, :, :] = final[None]

        for r in pending_sends:
            r.wait_send()

    return pl.pallas_call(
        body,
        out_shape=jax.ShapeDtypeStruct((1, SQ_SHARD, D), jnp.float32),
        in_specs=[
            pl.BlockSpec(memory_space=pltpu.VMEM),
            pl.BlockSpec(memory_space=pltpu.VMEM),
            pl.BlockSpec(memory_space=pltpu.VMEM),
            pl.BlockSpec(memory_space=pl.ANY),
            pl.BlockSpec(memory_space=pl.ANY),
        ],
        out_specs=pl.BlockSpec(memory_space=pltpu.VMEM),
        scratch_shapes=[
            pltpu.VMEM((N_DEV, SQ_SHARD, D), jnp.bfloat16),
            pltpu.VMEM((HQ_LOCAL, SKV, DH), jnp.bfloat16),
            pltpu.VMEM((HQ_LOCAL, SKV, DH), jnp.bfloat16),
            pltpu.VMEM((8, SKV, DH), jnp.float32),
            pltpu.VMEM((SQ_SHARD, D), jnp.float32),
            pltpu.VMEM((N_DEV - 1, SQ_SHARD, D), jnp.bfloat16),
            pltpu.VMEM((N_DEV - 1, SQ_SHARD, D), jnp.bfloat16),
            pltpu.SemaphoreType.DMA((N_DEV - 1,)),
            pltpu.SemaphoreType.DMA((N_DEV - 1,)),
            pltpu.SemaphoreType.DMA((N_DEV - 1,)),
            pltpu.SemaphoreType.DMA((N_DEV - 1,)),
            pltpu.SemaphoreType.DMA((8,)),
        ],
        compiler_params=pltpu.CompilerParams(
            collective_id=0,
            vmem_limit_bytes=100 * 1024 * 1024,
        ),
    )(x, Wq, Wo, K_ext, V_ext)


# device time: 66222 ns/iter; 2.7826x vs baseline; 1.0043x over previous
import jax
import jax.numpy as jnp
from jax import lax
from jax.experimental import pallas as pl
from jax.experimental.pallas import tpu as pltpu

N_DEV = 4
SQ_SHARD = 256
SQ = 1024
D = 1024
HQ_LOCAL = 8
DH = 128
SKV = 4096
SCALE = 0.08838834764831843
SCALE2 = SCALE * 1.4426950408889634


def kernel(x, Wq, Wo, K_ext, V_ext):
    def body(
        x_ref, wq_ref, wo_ref, k_ref, v_ref, out_ref,
        xg_ref, kb_ref, vb_ref, stage_ref, pmine_ref, rs_send_ref, rs_recv_ref,
        ag_send_sems, ag_recv_sems, rs_send_sems, rs_recv_sems, stage_sems,
    ):
        my = lax.axis_index("i")
        left = lax.rem(my + N_DEV - 1, N_DEV)
        right = lax.rem(my + 1, N_DEV)

        xg_ref[pl.ds(my, 1), :, :] = x_ref[:, :, :].astype(jnp.bfloat16)

        def ag_copy(h, slot):
            return pltpu.make_async_remote_copy(
                src_ref=xg_ref.at[slot],
                dst_ref=xg_ref.at[slot],
                send_sem=ag_send_sems.at[h],
                recv_sem=ag_recv_sems.at[h],
                device_id=(right,),
                device_id_type=pl.DeviceIdType.MESH,
            )

        def rs_copy(s):
            return pltpu.make_async_remote_copy(
                src_ref=rs_send_ref.at[s],
                dst_ref=rs_recv_ref.at[s],
                send_sem=rs_send_sems.at[s],
                recv_sem=rs_recv_sems.at[s],
                device_id=(right,),
                device_id_type=pl.DeviceIdType.MESH,
            )

        pending_sends = []

        h0 = my * HQ_LOCAL

        def start_head_fetch(h):
            ks, vs = (2 * h) % 8, (2 * h + 1) % 8
            ck = pltpu.make_async_copy(
                k_ref.at[0, :, h0 + h, :], stage_ref.at[ks], stage_sems.at[ks]
            )
            cv = pltpu.make_async_copy(
                v_ref.at[0, :, h0 + h, :], stage_ref.at[vs], stage_sems.at[vs]
            )
            ck.start()
            cv.start()
            return ck, cv

        fetches = {h: start_head_fetch(h) for h in range(4)}

        barrier_sem = pltpu.get_barrier_semaphore()
        for nbr in (left, right):
            pl.semaphore_signal(
                barrier_sem, inc=1,
                device_id=(nbr,), device_id_type=pl.DeviceIdType.MESH,
            )
        pl.semaphore_wait(barrier_sem, 2)

        ag0 = ag_copy(0, my)
        ag0.start()
        pending_sends.append(ag0)

        wqb = wq_ref[:, :].astype(jnp.bfloat16)
        wob = wo_ref[:, :].astype(jnp.bfloat16)
        rs_rdmas = [None] * (N_DEV - 1)

        for j in range(N_DEV):
            sc = lax.rem(my - j + N_DEV, N_DEV)
            if j > 0:
                ag_copy(j - 1, sc).wait_recv()
                if j < N_DEV - 1:
                    fwd = ag_copy(j, sc)
                    fwd.start()
                    pending_sends.append(fwd)

            xc = xg_ref[pl.ds(sc, 1), :, :].reshape(SQ_SHARD, D)
            qc = (jnp.dot(
                xc, wqb, preferred_element_type=jnp.float32,
            ) * SCALE2).astype(jnp.bfloat16)
            head_outs = []
            for h in range(HQ_LOCAL):
                if j == 0:
                    ck, cv = fetches.pop(h)
                    ck.wait()
                    cv.wait()
                    kb_ref[h] = stage_ref[(2 * h) % 8].astype(jnp.bfloat16)
                    vb_ref[h] = stage_ref[(2 * h + 1) % 8].astype(jnp.bfloat16)
                    if h + 4 < HQ_LOCAL:
                        fetches[h + 4] = start_head_fetch(h + 4)
                qh = qc[:, h * DH:(h + 1) * DH]
                s_blk = lax.dot_general(
                    qh, kb_ref[h], (((1,), (1,)), ((), ())),
                    preferred_element_type=jnp.float32,
                )
                p = jnp.exp2(s_blk).astype(jnp.bfloat16)
                lh = jnp.sum(p, axis=1, keepdims=True, dtype=jnp.float32)
                pv = lax.dot_general(
                    p, vb_ref[h], (((1,), (0,)), ((), ())),
                    preferred_element_type=jnp.float32,
                )
                head_outs.append((pv / lh).astype(jnp.bfloat16))
            attn_c = jnp.concatenate(head_outs, axis=1)

            if j == 0:
                pmine_ref[:, :] = jnp.dot(
                    attn_c, wob, preferred_element_type=jnp.float32
                )
            elif j < N_DEV - 1:
                s = j - 1
                pc = jnp.dot(attn_c, wob, preferred_element_type=jnp.float32)
                if s > 0:
                    rs_rdmas[s - 1].wait_recv()
                    pc = pc + rs_recv_ref[s - 1].astype(jnp.float32)
                rs_send_ref[s] = pc.astype(jnp.bfloat16)
                rs = rs_copy(s)
                rs.start()
                rs_rdmas[s] = rs
                pending_sends.append(rs)
            else:
                s = j - 1
                rs_rdmas[s - 1].wait_recv()
                last_rdmas = []
                for hf in range(2):
                    r0 = hf * (SQ_SHARD // 2)
                    rows = slice(r0, r0 + SQ_SHARD // 2)
                    pch = jnp.dot(
                        attn_c[rows, :], wob,
                        preferred_element_type=jnp.float32,
                    ) + rs_recv_ref[s - 1, rows, :].astype(jnp.float32)
                    rs_send_ref[s, rows, :] = pch.astype(jnp.bfloat16)
                    rh = pltpu.make_async_remote_copy(
                        src_ref=rs_send_ref.at[s, pl.ds(r0, SQ_SHARD // 2)],
                        dst_ref=rs_recv_ref.at[s, pl.ds(r0, SQ_SHARD // 2)],
                        send_sem=rs_send_sems.at[s + hf],
                        recv_sem=rs_recv_sems.at[s + hf],
                        device_id=(right,),
                        device_id_type=pl.DeviceIdType.MESH,
                    )
                    rh.start()
                    last_rdmas.append(rh)
                    pending_sends.append(rh)

        for rh in last_rdmas:
            rh.wait_recv()
        final = pmine_ref[:, :] + rs_recv_ref[N_DEV - 2].astype(jnp.float32)
        out_ref[:, :, :] = final[None]

        for r in pending_sends:
            r.wait_send()

    return pl.pallas_call(
        body,
        out_shape=jax.ShapeDtypeStruct((1, SQ_SHARD, D), jnp.float32),
        in_specs=[
            pl.BlockSpec(memory_space=pltpu.VMEM),
            pl.BlockSpec(memory_space=pltpu.VMEM),
            pl.BlockSpec(memory_space=pltpu.VMEM),
            pl.BlockSpec(memory_space=pl.ANY),
            pl.BlockSpec(memory_space=pl.ANY),
        ],
        out_specs=pl.BlockSpec(memory_space=pltpu.VMEM),
        scratch_shapes=[
            pltpu.VMEM((N_DEV, SQ_SHARD, D), jnp.bfloat16),
            pltpu.VMEM((HQ_LOCAL, SKV, DH), jnp.bfloat16),
            pltpu.VMEM((HQ_LOCAL, SKV, DH), jnp.bfloat16),
            pltpu.VMEM((8, SKV, DH), jnp.float32),
            pltpu.VMEM((SQ_SHARD, D), jnp.float32),
            pltpu.VMEM((N_DEV - 1, SQ_SHARD, D), jnp.bfloat16),
            pltpu.VMEM((N_DEV - 1, SQ_SHARD, D), jnp.bfloat16),
            pltpu.SemaphoreType.DMA((N_DEV - 1,)),
            pltpu.SemaphoreType.DMA((N_DEV - 1,)),
            pltpu.SemaphoreType.DMA((N_DEV,)),
            pltpu.SemaphoreType.DMA((N_DEV,)),
            pltpu.SemaphoreType.DMA((8,)),
        ],
        compiler_params=pltpu.CompilerParams(
            collective_id=0,
            vmem_limit_bytes=100 * 1024 * 1024,
        ),
    )(x, Wq, Wo, K_ext, V_ext)
